# baseline (speedup 1.0000x reference)
import functools

import jax
import jax.numpy as jnp
from jax import lax
from jax.experimental import pallas as pl
from jax.experimental.pallas import tpu as pltpu
from jax.experimental.pallas import tpu_sc as plsc

_T = 64
_BT = 16

_NC, _NS, _L = 2, 16, 16
_NW = _NC * _NS


def _dense_step(x_ref, out_ref):
    x = x_ref[...]
    da = jnp.sum(x[..., :_T], axis=-1) - 1.0
    db = jnp.sum(x[..., _T:], axis=-1) - 1.0
    out_ref[...] = (jnp.sum(da * da) + jnp.sum(db * db)).reshape(1, 1, 1)


def _corr_body(x_hbm, ridx_hbm, par_hbm, wtab_hbm, out_hbm,
               idx_v, par_v, rows_v, w_v, acc_v, sem, sem2):
    bpw = ridx_hbm.shape[0] // _NW
    wid = lax.axis_index("s") * _NC + lax.axis_index("c")
    base = wid * bpw
    pltpu.sync_copy(ridx_hbm.at[pl.ds(base, bpw)], idx_v)
    pltpu.sync_copy(par_hbm.at[pl.ds(base, bpw)], par_v)
    pltpu.async_copy(x_hbm.at[idx_v], rows_v, sem).wait()
    pltpu.async_copy(wtab_hbm.at[par_v], w_v, sem2).wait()
    acc = jnp.zeros((_L,), jnp.float32)
    for r in range(bpw):
        for c in range(2 * _T // _L):
            sl = pl.ds(c * _L, _L)
            acc = acc + rows_v[r, sl] * w_v[r, sl]
    acc_v[...] = acc
    pltpu.sync_copy(acc_v, out_hbm.at[wid])


def kernel(outputs, labels):
    B, C, H, W, T = outputs.shape
    x = outputs.reshape(B, C // 2, 2 * T)
    n_steps = B // _BT
    dense = pl.pallas_call(
        _dense_step,
        grid=(n_steps,),
        in_specs=[pl.BlockSpec((_BT, C // 2, 2 * T), lambda i: (i, 0, 0))],
        out_specs=pl.BlockSpec((1, 1, 1), lambda i: (i, 0, 0)),
        out_shape=jax.ShapeDtypeStruct((n_steps, 1, 1), jnp.float32),
        compiler_params=pltpu.CompilerParams(
            dimension_semantics=("parallel",)),
    )(x)

    bpw = B // _NW
    ridx = (jnp.arange(B, dtype=jnp.int32) * C + labels) // 2
    par = labels & 1
    lane = jnp.arange(2 * T, dtype=jnp.int32)
    wtab = jnp.stack([(lane < T).astype(jnp.float32),
                      (lane >= T).astype(jnp.float32)])
    x2 = outputs.reshape(B * C // 2, 2 * T)
    mesh = plsc.VectorSubcoreMesh(
        core_axis_name="c", subcore_axis_name="s",
        num_cores=_NC, num_subcores=_NS)
    corr = pl.kernel(
        _corr_body,
        out_type=jax.ShapeDtypeStruct((_NW, _L), jnp.float32),
        mesh=mesh,
        scratch_types=[
            pltpu.VMEM((bpw,), jnp.int32),
            pltpu.VMEM((bpw,), jnp.int32),
            pltpu.VMEM((bpw, 2 * T), jnp.float32),
            pltpu.VMEM((bpw, 2 * T), jnp.float32),
            pltpu.VMEM((_L,), jnp.float32),
            pltpu.SemaphoreType.DMA,
            pltpu.SemaphoreType.DMA,
        ],
    )(x2, ridx, par, wtab)

    g = jnp.sum(corr)
    loss = (0.5 / T) * (jnp.sum(dense) + 99.0 * B - 18.0 * g)
    return loss


# speedup vs baseline: 10.6728x; 10.6728x over previous
import jax
import jax.numpy as jnp
from jax.experimental import pallas as pl
from jax.experimental.pallas import tpu as pltpu

_CC = 50


def _loss_step(lab_ref, x_ref, out_ref):
    x = x_ref[...]
    T = x.shape[1]
    s = jnp.sum(x, axis=1)
    d = s - 1.0
    part = jnp.sum(d * d)
    lab = lab_ref[0, :]
    c_idx = (jax.lax.broadcasted_iota(jnp.int32, s.shape, 0)
             + pl.program_id(0) * _CC)
    corr = jnp.sum(jnp.where(lab[None, :] == c_idx, 99.0 - 18.0 * s, 0.0))
    out_ref[...] = ((part + corr) * (0.5 / T)).reshape(1, 1, 1)


def kernel(outputs, labels):
    B, C, H, W, T = outputs.shape
    xt = jnp.transpose(outputs.reshape(B, C, T), (1, 2, 0))
    n_steps = C // _CC
    lab2 = labels.reshape(1, B)
    out = pl.pallas_call(
        _loss_step,
        grid=(n_steps,),
        in_specs=[
            pl.BlockSpec((1, B), lambda i: (0, 0)),
            pl.BlockSpec((_CC, T, B), lambda i: (i, 0, 0)),
        ],
        out_specs=pl.BlockSpec((1, 1, 1), lambda i: (i, 0, 0)),
        out_shape=jax.ShapeDtypeStruct((n_steps, 1, 1), jnp.float32),
        compiler_params=pltpu.CompilerParams(
            dimension_semantics=("parallel",)),
    )(lab2, xt)
    return jnp.sum(out)


# speedup vs baseline: 12.8583x; 1.2048x over previous
import jax
import jax.numpy as jnp
from jax.experimental import pallas as pl
from jax.experimental.pallas import tpu as pltpu

_CC = 125


def _loss_step(lab_ref, x_ref, out_ref):
    x = x_ref[...]
    T = x.shape[1]
    s = jnp.sum(x, axis=1)
    d = s - 1.0
    part = jnp.sum(d * d)
    lab = lab_ref[0, :]
    c_idx = (jax.lax.broadcasted_iota(jnp.int32, s.shape, 0)
             + pl.program_id(0) * _CC)
    corr = jnp.sum(jnp.where(lab[None, :] == c_idx, 99.0 - 18.0 * s, 0.0))
    out_ref[...] = ((part + corr) * (0.5 / T)).reshape(1, 1, 1)


def kernel(outputs, labels):
    B, C, H, W, T = outputs.shape
    xt = jnp.transpose(outputs.reshape(B, C, T), (1, 2, 0))
    n_steps = C // _CC
    lab2 = labels.reshape(1, B)
    out = pl.pallas_call(
        _loss_step,
        grid=(n_steps,),
        in_specs=[
            pl.BlockSpec((1, B), lambda i: (0, 0)),
            pl.BlockSpec((_CC, T, B), lambda i: (i, 0, 0)),
        ],
        out_specs=pl.BlockSpec((1, 1, 1), lambda i: (i, 0, 0)),
        out_shape=jax.ShapeDtypeStruct((n_steps, 1, 1), jnp.float32),
        compiler_params=pltpu.CompilerParams(
            dimension_semantics=("parallel",)),
    )(lab2, xt)
    return jnp.sum(out)
